# baseline (speedup 1.0000x reference)
import functools

import jax
import jax.numpy as jnp
from jax import lax
from jax.experimental import pallas as pl
from jax.experimental.pallas import tpu as pltpu
from jax.experimental.pallas import tpu_sc as plsc

_NC, _NS, _L = 2, 16, 16
_NW = _NC * _NS


def _sc_edge_count(e0, e1, zeros_flat, n):
    m = e0.shape[0]
    per_w = m // _NS
    ndma = per_w // 128
    mesh = plsc.VectorSubcoreMesh(core_axis_name="c", subcore_axis_name="s",
                                  num_cores=1)

    @functools.partial(
        pl.kernel, mesh=mesh,
        compiler_params=pltpu.CompilerParams(needs_layout_passes=False),
        out_type=jax.ShapeDtypeStruct((1, n * n), jnp.float32),
        scratch_types=[
            pltpu.VMEM_SHARED((n * n + n,), jnp.float32),
            pltpu.VMEM((per_w,), jnp.int32),
            pltpu.VMEM((per_w,), jnp.int32),
            pltpu.VMEM((ndma, 128), jnp.int32),
            pltpu.VMEM((ndma, 128), jnp.float32),
        ],
    )
    def edge_count(e0_hbm, e1_hbm, zero_hbm, out_hbm,
                   c_sh, e0_v, e1_v, idx_v, val_v):
        cid = lax.axis_index("c")
        sid = lax.axis_index("s")
        base = sid * per_w
        pltpu.sync_copy(e0_hbm.at[pl.ds(base, per_w)], e0_v)
        pltpu.sync_copy(e1_hbm.at[pl.ds(base, per_w)], e1_v)

        @pl.when(sid == 0)
        def _zero():
            pltpu.sync_copy(zero_hbm, c_sh.at[pl.ds(0, n * n)])

        ones = jnp.ones((_L,), jnp.float32)
        for j in range(ndma):
            for t in range(128 // _L):
                off = j * 128 + t * _L
                flat = e0_v[pl.ds(off, _L)] * n + e1_v[pl.ds(off, _L)]
                idx_v[j, pl.ds(t * _L, _L)] = flat
                val_v[j, pl.ds(t * _L, _L)] = ones
        plsc.subcore_barrier()
        for j in range(ndma):
            pltpu.sync_copy(val_v.at[j], c_sh.at[idx_v.at[j]], add=True)
        plsc.subcore_barrier()

        @pl.when(sid == 0)
        def _out():
            pltpu.sync_copy(c_sh.at[pl.ds(0, n * n)], out_hbm.at[cid])

    return edge_count(e0, e1, zeros_flat)


def _onehot(idx_col, rows, n, dtype):
    iota = lax.broadcasted_iota(jnp.int16, (rows, n), 1)
    return jnp.where(idx_col.astype(jnp.int16) == iota,
                     jnp.array(1, dtype), jnp.array(0, dtype))


def _tc_body(tri_ref, g_ref, p_ref, beta_ref, tri_out_ref, logg_ref,
             h_ref, acc_ref, *, B, N, K, BLK, nblocks):
    i = pl.program_id(0)
    beta = beta_ref[0, 0]

    @pl.when(i == 0)
    def _init():
        for bb in range(B):
            hb = jnp.dot(g_ref[bb], p_ref[...],
                         preferred_element_type=jnp.float32)
            h_ref[:, bb * K:(bb + 1) * K] = (
                (beta * 1.4426950408889634) * hb).astype(jnp.bfloat16)
            eb = jnp.exp(beta * hb).astype(jnp.bfloat16)
            gb = lax.dot_general(eb, eb, (((1,), (1,)), ((), ())),
                                 preferred_element_type=jnp.float32)
            logg_ref[bb] = jnp.log(gb)
        acc_ref[0] = 0.0
        acc_ref[1] = jnp.sum(g_ref[...] ** 2)

    tidx = tri_ref[...]
    at = (_onehot(tidx[:, 0:1], BLK, N, jnp.bfloat16)
          + _onehot(tidx[:, 1:2], BLK, N, jnp.bfloat16)
          + _onehot(tidx[:, 2:3], BLK, N, jnp.bfloat16))
    valid = tidx[:, 0:1] >= 0
    hs = jnp.dot(at, h_ref[...], preferred_element_type=jnp.float32)
    prod = jnp.float32(1.0)
    for bb in range(B):
        s = jnp.sum(jnp.exp2(hs[:, bb * K:(bb + 1) * K]),
                    axis=1, keepdims=True)
        prod = prod * s
    acc_ref[0] += jnp.sum(jnp.where(valid, jnp.log(prod), 0.0))

    @pl.when(i == nblocks - 1)
    def _fin():
        tri_out_ref[...] = jnp.reshape(
            jnp.stack([acc_ref[0], acc_ref[1]]), (1, 2))


def _combine_body(cmat_ref, logg_ref, tri_ref, beta_ref, out_ref,
                  *, B, N, num_simplices):
    beta = beta_ref[0, 0]
    c = jnp.reshape(cmat_ref[0], (N, N))
    edge_lse = jnp.float32(0.0)
    for bb in range(B):
        edge_lse = edge_lse + jnp.sum(c * logg_ref[bb])
    total_lse = tri_ref[0, 0] + edge_lse
    energy = (-(1.0 / (beta * num_simplices)) * total_lse
              - 2.0 * tri_ref[0, 1]) / (B * N)
    out_ref[...] = jnp.reshape(energy, (1, 1))


def kernel(g, patterns, beta, edges, triangles):
    B, N, D = g.shape
    K = patterns.shape[1]
    m2, m3 = edges.shape[0], triangles.shape[0]
    num_simplices = m2 + m3

    edges = edges.astype(jnp.int32)
    triangles = triangles.astype(jnp.int32)

    m2_pad = ((m2 + 128 * _NS - 1) // (128 * _NS)) * (128 * _NS)
    e0 = jnp.pad(edges[:, 0], (0, m2_pad - m2), constant_values=N)
    e1 = jnp.pad(edges[:, 1], (0, m2_pad - m2), constant_values=0)
    zeros_flat = jnp.zeros((N * N,), jnp.float32)
    cmat = _sc_edge_count(e0, e1, zeros_flat, N)

    BLK = 4096
    nblocks = (m3 + BLK - 1) // BLK
    triangles = jnp.pad(triangles, ((0, nblocks * BLK - m3), (0, 0)),
                        constant_values=-1)
    beta_arr = jnp.reshape(beta.astype(jnp.float32), (1, 1))

    body = functools.partial(_tc_body, B=B, N=N, K=K, BLK=BLK,
                             nblocks=nblocks)
    tri_out, logg = pl.pallas_call(
        body,
        grid=(nblocks,),
        in_specs=[
            pl.BlockSpec((BLK, 3), lambda i: (i, 0)),
            pl.BlockSpec((B, N, D), lambda i: (0, 0, 0)),
            pl.BlockSpec((D, K), lambda i: (0, 0)),
            pl.BlockSpec((1, 1), lambda i: (0, 0)),
        ],
        out_specs=[
            pl.BlockSpec((1, 2), lambda i: (0, 0)),
            pl.BlockSpec((B, N, N), lambda i: (0, 0, 0)),
        ],
        out_shape=[
            jax.ShapeDtypeStruct((1, 2), jnp.float32),
            jax.ShapeDtypeStruct((B, N, N), jnp.float32),
        ],
        scratch_shapes=[
            pltpu.VMEM((N, B * K), jnp.bfloat16),
            pltpu.SMEM((2,), jnp.float32),
        ],
    )(triangles, g, patterns, beta_arr)

    combine = functools.partial(_combine_body, B=B, N=N,
                                num_simplices=num_simplices)
    out = pl.pallas_call(
        combine,
        out_shape=jax.ShapeDtypeStruct((1, 1), jnp.float32),
    )(cmat, logg, tri_out, beta_arr)
    return jnp.reshape(out, ())


# speedup vs baseline: 1.1610x; 1.1610x over previous
import functools

import jax
import jax.numpy as jnp
from jax import lax
from jax.experimental import pallas as pl
from jax.experimental.pallas import tpu as pltpu
from jax.experimental.pallas import tpu_sc as plsc

_NC, _NS, _L = 2, 16, 16
_NW = _NC * _NS


def _sc_edge_count(e0, e1, zeros_flat, n):
    m = e0.shape[0]
    per_w = m // _NS
    ndma = per_w // 128
    mesh = plsc.VectorSubcoreMesh(core_axis_name="c", subcore_axis_name="s",
                                  num_cores=1)

    @functools.partial(
        pl.kernel, mesh=mesh,
        compiler_params=pltpu.CompilerParams(needs_layout_passes=False),
        out_type=jax.ShapeDtypeStruct((1, n * n), jnp.float32),
        scratch_types=[
            pltpu.VMEM_SHARED((n * n + n,), jnp.float32),
            pltpu.VMEM((per_w,), jnp.int32),
            pltpu.VMEM((per_w,), jnp.int32),
            pltpu.VMEM((ndma, 128), jnp.int32),
            pltpu.VMEM((ndma, 128), jnp.float32),
        ],
    )
    def edge_count(e0_hbm, e1_hbm, zero_hbm, out_hbm,
                   c_sh, e0_v, e1_v, idx_v, val_v):
        cid = lax.axis_index("c")
        sid = lax.axis_index("s")
        base = sid * per_w
        pltpu.sync_copy(e0_hbm.at[pl.ds(base, per_w)], e0_v)
        pltpu.sync_copy(e1_hbm.at[pl.ds(base, per_w)], e1_v)

        @pl.when(sid == 0)
        def _zero():
            pltpu.sync_copy(zero_hbm, c_sh.at[pl.ds(0, n * n)])

        ones = jnp.ones((_L,), jnp.float32)
        for j in range(ndma):
            for t in range(128 // _L):
                off = j * 128 + t * _L
                flat = e0_v[pl.ds(off, _L)] * n + e1_v[pl.ds(off, _L)]
                idx_v[j, pl.ds(t * _L, _L)] = flat
                val_v[j, pl.ds(t * _L, _L)] = ones
        plsc.subcore_barrier()
        for j in range(ndma):
            pltpu.sync_copy(val_v.at[j], c_sh.at[idx_v.at[j]], add=True)
        plsc.subcore_barrier()

        @pl.when(sid == 0)
        def _out():
            pltpu.sync_copy(c_sh.at[pl.ds(0, n * n)], out_hbm.at[cid])

    return edge_count(e0, e1, zeros_flat)


def _onehot(idx_col, rows, n, dtype):
    iota = lax.broadcasted_iota(jnp.int16, (rows, n), 1)
    return jnp.where(idx_col.astype(jnp.int16) == iota,
                     jnp.array(1, dtype), jnp.array(0, dtype))


def _tc_body(tri_ref, g_ref, p_ref, beta_ref, cmat_ref, out_ref,
             h_ref, logg_ref, acc_ref, *, B, N, K, BLK, nblocks,
             num_simplices, padded):
    i = pl.program_id(0)
    beta = beta_ref[0, 0]

    @pl.when(i == 0)
    def _init():
        for bb in range(B):
            hb = jnp.dot(g_ref[bb], p_ref[...],
                         preferred_element_type=jnp.float32)
            h_ref[:, bb * K:(bb + 1) * K] = (
                (beta * 1.4426950408889634) * hb).astype(jnp.bfloat16)
            eb = jnp.exp(beta * hb).astype(jnp.bfloat16)
            gb = lax.dot_general(eb, eb, (((1,), (1,)), ((), ())),
                                 preferred_element_type=jnp.float32)
            logg_ref[bb] = jnp.log(gb)
        acc_ref[0] = 0.0
        acc_ref[1] = jnp.sum(g_ref[...] ** 2)

    tidx = tri_ref[...]
    at = (_onehot(tidx[:, 0:1], BLK, N, jnp.bfloat16)
          + _onehot(tidx[:, 1:2], BLK, N, jnp.bfloat16)
          + _onehot(tidx[:, 2:3], BLK, N, jnp.bfloat16))
    hs = jnp.dot(at, h_ref[...], preferred_element_type=jnp.float32)
    prod = jnp.float32(1.0)
    for bb in range(B):
        s = jnp.sum(jnp.exp2(hs[:, bb * K:(bb + 1) * K]),
                    axis=1, keepdims=True)
        prod = prod * s
    if padded:
        valid = tidx[:, 0:1] >= 0
        acc_ref[0] += jnp.sum(jnp.where(valid, jnp.log(prod), 0.0))
    else:
        acc_ref[0] += jnp.sum(jnp.log(prod))

    @pl.when(i == nblocks - 1)
    def _fin():
        c = jnp.reshape(cmat_ref[0], (N, N))
        edge_lse = jnp.float32(0.0)
        for bb in range(B):
            edge_lse = edge_lse + jnp.sum(c * logg_ref[bb])
        total_lse = acc_ref[0] + edge_lse
        energy = (-(1.0 / (beta * num_simplices)) * total_lse
                  - 2.0 * acc_ref[1]) / (B * N)
        out_ref[...] = jnp.reshape(energy, (1, 1))


def kernel(g, patterns, beta, edges, triangles):
    B, N, D = g.shape
    K = patterns.shape[1]
    m2, m3 = edges.shape[0], triangles.shape[0]
    num_simplices = m2 + m3

    edges = edges.astype(jnp.int32)
    triangles = triangles.astype(jnp.int32)

    m2_pad = ((m2 + 128 * _NS - 1) // (128 * _NS)) * (128 * _NS)
    e0 = jnp.pad(edges[:, 0], (0, m2_pad - m2), constant_values=N)
    e1 = jnp.pad(edges[:, 1], (0, m2_pad - m2), constant_values=0)
    zeros_flat = jnp.zeros((N * N,), jnp.float32)
    cmat = _sc_edge_count(e0, e1, zeros_flat, N)

    BLK = 4080 if m3 % 4080 == 0 else 4096
    nblocks = (m3 + BLK - 1) // BLK
    triangles = jnp.pad(triangles, ((0, nblocks * BLK - m3), (0, 0)),
                        constant_values=-1)
    beta_arr = jnp.reshape(beta.astype(jnp.float32), (1, 1))

    body = functools.partial(_tc_body, B=B, N=N, K=K, BLK=BLK,
                             nblocks=nblocks, num_simplices=num_simplices,
                             padded=nblocks * BLK > m3)
    out = pl.pallas_call(
        body,
        grid=(nblocks,),
        in_specs=[
            pl.BlockSpec((BLK, 3), lambda i: (i, 0)),
            pl.BlockSpec((B, N, D), lambda i: (0, 0, 0)),
            pl.BlockSpec((D, K), lambda i: (0, 0)),
            pl.BlockSpec((1, 1), lambda i: (0, 0)),
            pl.BlockSpec((1, N * N), lambda i: (0, 0)),
        ],
        out_specs=pl.BlockSpec((1, 1), lambda i: (0, 0)),
        out_shape=jax.ShapeDtypeStruct((1, 1), jnp.float32),
        scratch_shapes=[
            pltpu.VMEM((N, B * K), jnp.bfloat16),
            pltpu.VMEM((B, N, N), jnp.float32),
            pltpu.SMEM((2,), jnp.float32),
        ],
    )(triangles, g, patterns, beta_arr, cmat)
    return jnp.reshape(out, ())
